# baseline (speedup 1.0000x reference)
import functools

import jax
import jax.numpy as jnp
from jax import lax
from jax.experimental import pallas as pl
from jax.experimental.pallas import tpu as pltpu
from jax.experimental.pallas import tpu_sc as plsc

N = 10000
D = 128
E = 320000
NP = 10240
NC, NS = 2, 16
NW = NC * NS
EPT = E // NW
K = 80
CH = EPT // K
NB = 5
RPT = NP // NS
DW = 8

_MESH = plsc.VectorSubcoreMesh(
    core_axis_name="c", subcore_axis_name="s", num_cores=NC, num_subcores=NS)


@functools.partial(
    pl.kernel,
    out_type=jax.ShapeDtypeStruct((NC, NP, DW), jnp.float32),
    mesh=_MESH,
    compiler_params=pltpu.CompilerParams(use_tc_tiling_on_sc=False),
    scratch_types=[
        pltpu.VMEM((CH, K), jnp.int32),
        pltpu.VMEM((K, DW), jnp.float32),
        pltpu.VMEM_SHARED((NP, DW), jnp.float32),
    ],
)
def _sc_degree(dst_hbm, ones_hbm, zeros_hbm, out_hbm, idx_v, ones_v, deg_sh):
    c = lax.axis_index("c")
    s = lax.axis_index("s")
    wid = c * NS + s
    pltpu.sync_copy(zeros_hbm.at[pl.ds(s * RPT, RPT)],
                    deg_sh.at[pl.ds(s * RPT, RPT)])
    pltpu.sync_copy(dst_hbm.at[wid], idx_v)
    pltpu.sync_copy(ones_hbm, ones_v)
    plsc.subcore_barrier()

    @pl.loop(0, CH)
    def _chunk(j):
        pltpu.sync_copy(ones_v, deg_sh.at[idx_v.at[j]], add=True)

    plsc.subcore_barrier()
    pltpu.sync_copy(deg_sh.at[pl.ds(s * RPT, RPT)],
                    out_hbm.at[c, pl.ds(s * RPT, RPT)])


DH = D // 2


@functools.partial(
    pl.kernel,
    out_type=jax.ShapeDtypeStruct((NC, 2, NP, DH), jnp.float32),
    mesh=_MESH,
    compiler_params=pltpu.CompilerParams(use_tc_tiling_on_sc=False),
    scratch_types=(
        [
            pltpu.VMEM((CH, K), jnp.int32),
            pltpu.VMEM((CH, K), jnp.int32),
            pltpu.VMEM_SHARED((NP, DH), jnp.float32),
        ]
        + [pltpu.VMEM((K, DH), jnp.float32) for _ in range(NB)]
        + [pltpu.SemaphoreType.DMA for _ in range(2 * NB)]
    ),
)
def _sc_aggregate(y0_hbm, y1_hbm, src_hbm, dst_hbm, zeros_hbm, out_hbm,
                  src_v, dst_v, acc_sh, *bufs_and_sems):
    bufs = bufs_and_sems[:NB]
    gsem = bufs_and_sems[NB:2 * NB]
    ssem = bufs_and_sems[2 * NB:]
    c = lax.axis_index("c")
    s = lax.axis_index("s")
    wid = c * NS + s
    pltpu.sync_copy(src_hbm.at[wid], src_v)
    pltpu.sync_copy(dst_hbm.at[wid], dst_v)

    for half, y_hbm in ((0, y0_hbm), (1, y1_hbm)):
        pltpu.sync_copy(zeros_hbm.at[pl.ds(s * RPT, RPT)],
                        acc_sh.at[pl.ds(s * RPT, RPT)])
        plsc.subcore_barrier()

        for b in range(NB):
            pltpu.async_copy(y_hbm.at[src_v.at[b]], bufs[b], gsem[b])

        @pl.loop(0, CH, step=NB)
        def _round(j):
            for b in range(NB):
                pltpu.make_async_copy(y_hbm.at[src_v.at[j + b]], bufs[b],
                                      gsem[b]).wait()
                pltpu.async_copy(bufs[b], acc_sh.at[dst_v.at[j + b]],
                                 ssem[b], add=True)
            for b in range(NB):
                @pl.when(j + b + NB < CH)
                def _refill():
                    pltpu.make_async_copy(bufs[b],
                                          acc_sh.at[dst_v.at[j + b]],
                                          ssem[b]).wait()
                    pltpu.async_copy(y_hbm.at[src_v.at[j + b + NB]],
                                     bufs[b], gsem[b])

        for b in range(NB):
            pltpu.make_async_copy(bufs[b], acc_sh.at[dst_v.at[CH - NB + b]],
                                  ssem[b]).wait()

        plsc.subcore_barrier()
        pltpu.sync_copy(acc_sh.at[pl.ds(s * RPT, RPT)],
                        out_hbm.at[c, half, pl.ds(s * RPT, RPT)])


_BLK = 2000


def _dot(a, b):
    return jnp.dot(a, b, preferred_element_type=jnp.float32,
                   precision=lax.Precision.HIGHEST)


def _tc_mm_body(x_ref, w1_ref, xw1_ref):
    xw1_ref[...] = _dot(x_ref[...], w1_ref[...])


def _tc1_body(degp_ref0, degp_ref1, xw_ref, y_ref, dinv_ref):
    deg = degp_ref0[0, :, 0:1] + degp_ref1[0, :, 0:1] + 1.0
    dinv = lax.rsqrt(deg)
    y_ref[...] = xw_ref[...] * dinv
    dinv_ref[...] = dinv


def _agg_h(a0a_ref, a0b_ref, a1a_ref, a1b_ref, xw_ref, dinv, b_ref):
    agg = jnp.concatenate(
        [a0a_ref[0, 0] + a1a_ref[0, 0], a0b_ref[0, 0] + a1b_ref[0, 0]],
        axis=1)
    h = dinv * agg + (dinv * dinv) * xw_ref[...]
    return jnp.maximum(h + b_ref[...], 0.0)


def _tc_mid_body(a0a_ref, a0b_ref, a1a_ref, a1b_ref, xw_ref, dinv_ref,
                 b_ref, w_ref, y_ref, xwn_ref):
    dinv = dinv_ref[...]
    h = _agg_h(a0a_ref, a0b_ref, a1a_ref, a1b_ref, xw_ref, dinv, b_ref)
    xwn = _dot(h, w_ref[...])
    xwn_ref[...] = xwn
    y_ref[...] = xwn * dinv


def _tc_out_body(a0a_ref, a0b_ref, a1a_ref, a1b_ref, xw_ref, dinv_ref,
                 b_ref, wfc_ref, bfc_ref, out_ref):
    dinv = dinv_ref[...]
    h = _agg_h(a0a_ref, a0b_ref, a1a_ref, a1b_ref, xw_ref, dinv, b_ref)
    out_ref[...] = _dot(h, wfc_ref[...]) + bfc_ref[...]


def _rows(i):
    return (i, 0)


def _fixed(i):
    return (0, 0)


def _degp_spec(core):
    return pl.BlockSpec((1, _BLK, DW), lambda i, c=core: (c, i, 0))


def _acc_spec(core, half):
    return pl.BlockSpec((1, 1, _BLK, DH),
                        lambda i, c=core, h=half: (c, h, i, 0))


_tc_mm = pl.pallas_call(
    _tc_mm_body,
    grid=(N // _BLK,),
    in_specs=[
        pl.BlockSpec((_BLK, D), _rows),
        pl.BlockSpec((D, D), _fixed),
    ],
    out_specs=pl.BlockSpec((_BLK, D), _rows),
    out_shape=jax.ShapeDtypeStruct((N, D), jnp.float32),
)

_tc1 = pl.pallas_call(
    _tc1_body,
    grid=(N // _BLK,),
    in_specs=[
        _degp_spec(0),
        _degp_spec(1),
        pl.BlockSpec((_BLK, D), _rows),
    ],
    out_specs=[
        pl.BlockSpec((_BLK, D), _rows),
        pl.BlockSpec((_BLK, 1), _rows),
    ],
    out_shape=[
        jax.ShapeDtypeStruct((N, D), jnp.float32),
        jax.ShapeDtypeStruct((N, 1), jnp.float32),
    ],
)

_tc_mid = pl.pallas_call(
    _tc_mid_body,
    grid=(N // _BLK,),
    in_specs=[
        _acc_spec(0, 0),
        _acc_spec(0, 1),
        _acc_spec(1, 0),
        _acc_spec(1, 1),
        pl.BlockSpec((_BLK, D), _rows),
        pl.BlockSpec((_BLK, 1), _rows),
        pl.BlockSpec((1, D), _fixed),
        pl.BlockSpec((D, D), _fixed),
    ],
    out_specs=[
        pl.BlockSpec((_BLK, D), _rows),
        pl.BlockSpec((_BLK, D), _rows),
    ],
    out_shape=[
        jax.ShapeDtypeStruct((N, D), jnp.float32),
        jax.ShapeDtypeStruct((N, D), jnp.float32),
    ],
)

_OUT = 40

_tc_out = pl.pallas_call(
    _tc_out_body,
    grid=(N // _BLK,),
    in_specs=[
        _acc_spec(0, 0),
        _acc_spec(0, 1),
        _acc_spec(1, 0),
        _acc_spec(1, 1),
        pl.BlockSpec((_BLK, D), _rows),
        pl.BlockSpec((_BLK, 1), _rows),
        pl.BlockSpec((1, D), _fixed),
        pl.BlockSpec((D, _OUT), _fixed),
        pl.BlockSpec((1, _OUT), _fixed),
    ],
    out_specs=pl.BlockSpec((_BLK, _OUT), _rows),
    out_shape=jax.ShapeDtypeStruct((N, _OUT), jnp.float32),
)


def kernel(x, edge_index, W1, b1, W2, b2, Wfc, bfc):
    src = edge_index[0].astype(jnp.int32).reshape(NW, CH, K)
    dst = edge_index[1].astype(jnp.int32).reshape(NW, CH, K)
    ones8 = jnp.ones((K, DW), jnp.float32)
    zeros_deg = jnp.zeros((NP, DW), jnp.float32)
    zeros_acc = jnp.zeros((NP, DH), jnp.float32)

    deg_p = _sc_degree(dst, ones8, zeros_deg)
    xw1 = _tc_mm(x, W1)

    y1, dinv = _tc1(deg_p, deg_p, xw1)

    acc1 = _sc_aggregate(y1[:, :DH], y1[:, DH:], src, dst, zeros_acc)
    y2, xw2 = _tc_mid(acc1, acc1, acc1, acc1, xw1, dinv,
                      b1.reshape(1, D), W2)

    acc2 = _sc_aggregate(y2[:, :DH], y2[:, DH:], src, dst, zeros_acc)
    return _tc_out(acc2, acc2, acc2, acc2, xw2, dinv,
                   b2.reshape(1, D), Wfc, bfc.reshape(1, _OUT))


# speedup vs baseline: 1.1102x; 1.1102x over previous
import functools

import jax
import jax.numpy as jnp
from jax import lax
from jax.experimental import pallas as pl
from jax.experimental.pallas import tpu as pltpu
from jax.experimental.pallas import tpu_sc as plsc

N = 10000
D = 128
E = 320000
NP = 10240
NC, NS = 2, 16
NW = NC * NS
EPT = E // NW
K = 80
CH = EPT // K
NB = 5
RPT = NP // NS
DW = 8

_MESH = plsc.VectorSubcoreMesh(
    core_axis_name="c", subcore_axis_name="s", num_cores=NC, num_subcores=NS)


@functools.partial(
    pl.kernel,
    out_type=jax.ShapeDtypeStruct((NC, NP, DW), jnp.float32),
    mesh=_MESH,
    compiler_params=pltpu.CompilerParams(use_tc_tiling_on_sc=False),
    scratch_types=[
        pltpu.VMEM((CH, K), jnp.int32),
        pltpu.VMEM((K, DW), jnp.float32),
        pltpu.VMEM_SHARED((NP, DW), jnp.float32),
    ],
)
def _sc_degree(dst_hbm, ones_hbm, zeros_hbm, out_hbm, idx_v, ones_v, deg_sh):
    c = lax.axis_index("c")
    s = lax.axis_index("s")
    wid = c * NS + s
    pltpu.sync_copy(zeros_hbm.at[pl.ds(s * RPT, RPT)],
                    deg_sh.at[pl.ds(s * RPT, RPT)])
    pltpu.sync_copy(dst_hbm.at[wid], idx_v)
    pltpu.sync_copy(ones_hbm, ones_v)
    plsc.subcore_barrier()

    @pl.loop(0, CH)
    def _chunk(j):
        pltpu.sync_copy(ones_v, deg_sh.at[idx_v.at[j]], add=True)

    plsc.subcore_barrier()
    pltpu.sync_copy(deg_sh.at[pl.ds(s * RPT, RPT)],
                    out_hbm.at[c, pl.ds(s * RPT, RPT)])


DH = D // 2


@functools.partial(
    pl.kernel,
    out_type=jax.ShapeDtypeStruct((NC, NP, D), jnp.float32),
    mesh=_MESH,
    compiler_params=pltpu.CompilerParams(use_tc_tiling_on_sc=False),
    scratch_types=(
        [
            pltpu.VMEM((CH, K), jnp.int32),
            pltpu.VMEM((CH, K), jnp.int32),
            pltpu.VMEM_SHARED((NP, DH), jnp.float32),
        ]
        + [pltpu.VMEM((K, DH), jnp.float32) for _ in range(NB)]
        + [pltpu.SemaphoreType.DMA for _ in range(2 * NB)]
    ),
)
def _sc_aggregate(y0_hbm, y1_hbm, src_hbm, dst_hbm, zeros_hbm, out_hbm,
                  src_v, dst_v, acc_sh, *bufs_and_sems):
    bufs = bufs_and_sems[:NB]
    gsem = bufs_and_sems[NB:2 * NB]
    ssem = bufs_and_sems[2 * NB:]
    c = lax.axis_index("c")
    s = lax.axis_index("s")
    wid = c * NS + s
    pltpu.sync_copy(src_hbm.at[wid], src_v)
    pltpu.sync_copy(dst_hbm.at[wid], dst_v)

    for half, y_hbm in ((0, y0_hbm), (1, y1_hbm)):
        pltpu.sync_copy(zeros_hbm.at[pl.ds(s * RPT, RPT)],
                        acc_sh.at[pl.ds(s * RPT, RPT)])
        plsc.subcore_barrier()

        for b in range(NB):
            pltpu.async_copy(y_hbm.at[src_v.at[b]], bufs[b], gsem[b])

        @pl.loop(0, CH, step=NB)
        def _round(j):
            for b in range(NB):
                pltpu.make_async_copy(y_hbm.at[src_v.at[j + b]], bufs[b],
                                      gsem[b]).wait()
                pltpu.async_copy(bufs[b], acc_sh.at[dst_v.at[j + b]],
                                 ssem[b], add=True)
            for b in range(NB):
                @pl.when(j + b + NB < CH)
                def _refill():
                    pltpu.make_async_copy(bufs[b],
                                          acc_sh.at[dst_v.at[j + b]],
                                          ssem[b]).wait()
                    pltpu.async_copy(y_hbm.at[src_v.at[j + b + NB]],
                                     bufs[b], gsem[b])

        for b in range(NB):
            pltpu.make_async_copy(bufs[b], acc_sh.at[dst_v.at[CH - NB + b]],
                                  ssem[b]).wait()

        plsc.subcore_barrier()
        pltpu.sync_copy(acc_sh.at[pl.ds(s * RPT, RPT)],
                        out_hbm.at[c, pl.ds(s * RPT, RPT),
                                   pl.ds(half * DH, DH)])


_BLK = 2000


def _dot(a, b):
    return jnp.dot(a, b, preferred_element_type=jnp.float32,
                   precision=lax.Precision.HIGHEST)


def _tc_mm_body(x_ref, w1_ref, xw1_ref):
    xw1_ref[...] = _dot(x_ref[...], w1_ref[...])


def _tc1_body(degp_ref0, degp_ref1, xw_ref, y_ref, dinv_ref):
    deg = degp_ref0[0, :, 0:1] + degp_ref1[0, :, 0:1] + 1.0
    dinv = lax.rsqrt(deg)
    y_ref[...] = xw_ref[...] * dinv
    dinv_ref[...] = dinv


def _agg_h(a0_ref, a1_ref, xw_ref, dinv, b_ref):
    h = dinv * (a0_ref[0] + a1_ref[0]) + (dinv * dinv) * xw_ref[...]
    return jnp.maximum(h + b_ref[...], 0.0)


def _tc_mid_body(a0_ref, a1_ref, xw_ref, dinv_ref, b_ref, w_ref,
                 y_ref, xwn_ref):
    dinv = dinv_ref[...]
    h = _agg_h(a0_ref, a1_ref, xw_ref, dinv, b_ref)
    xwn = _dot(h, w_ref[...])
    xwn_ref[...] = xwn
    y_ref[...] = xwn * dinv


def _tc_out_body(a0_ref, a1_ref, xw_ref, dinv_ref, b_ref, wfc_ref, bfc_ref,
                 out_ref):
    dinv = dinv_ref[...]
    h = _agg_h(a0_ref, a1_ref, xw_ref, dinv, b_ref)
    out_ref[...] = _dot(h, wfc_ref[...]) + bfc_ref[...]


def _rows(i):
    return (i, 0)


def _fixed(i):
    return (0, 0)


def _degp_spec(core):
    return pl.BlockSpec((1, _BLK, DW), lambda i, c=core: (c, i, 0))


def _acc_spec(core):
    return pl.BlockSpec((1, _BLK, D), lambda i, c=core: (c, i, 0))


_tc_mm = pl.pallas_call(
    _tc_mm_body,
    grid=(N // _BLK,),
    in_specs=[
        pl.BlockSpec((_BLK, D), _rows),
        pl.BlockSpec((D, D), _fixed),
    ],
    out_specs=pl.BlockSpec((_BLK, D), _rows),
    out_shape=jax.ShapeDtypeStruct((N, D), jnp.float32),
)

_tc1 = pl.pallas_call(
    _tc1_body,
    grid=(N // _BLK,),
    in_specs=[
        _degp_spec(0),
        _degp_spec(1),
        pl.BlockSpec((_BLK, D), _rows),
    ],
    out_specs=[
        pl.BlockSpec((_BLK, D), _rows),
        pl.BlockSpec((_BLK, 1), _rows),
    ],
    out_shape=[
        jax.ShapeDtypeStruct((N, D), jnp.float32),
        jax.ShapeDtypeStruct((N, 1), jnp.float32),
    ],
)

_tc_mid = pl.pallas_call(
    _tc_mid_body,
    grid=(N // _BLK,),
    in_specs=[
        _acc_spec(0),
        _acc_spec(1),
        pl.BlockSpec((_BLK, D), _rows),
        pl.BlockSpec((_BLK, 1), _rows),
        pl.BlockSpec((1, D), _fixed),
        pl.BlockSpec((D, D), _fixed),
    ],
    out_specs=[
        pl.BlockSpec((_BLK, D), _rows),
        pl.BlockSpec((_BLK, D), _rows),
    ],
    out_shape=[
        jax.ShapeDtypeStruct((N, D), jnp.float32),
        jax.ShapeDtypeStruct((N, D), jnp.float32),
    ],
)

_OUT = 40

_tc_out = pl.pallas_call(
    _tc_out_body,
    grid=(N // _BLK,),
    in_specs=[
        _acc_spec(0),
        _acc_spec(1),
        pl.BlockSpec((_BLK, D), _rows),
        pl.BlockSpec((_BLK, 1), _rows),
        pl.BlockSpec((1, D), _fixed),
        pl.BlockSpec((D, _OUT), _fixed),
        pl.BlockSpec((1, _OUT), _fixed),
    ],
    out_specs=pl.BlockSpec((_BLK, _OUT), _rows),
    out_shape=jax.ShapeDtypeStruct((N, _OUT), jnp.float32),
)


def kernel(x, edge_index, W1, b1, W2, b2, Wfc, bfc):
    src = edge_index[0].astype(jnp.int32).reshape(NW, CH, K)
    dst = edge_index[1].astype(jnp.int32).reshape(NW, CH, K)
    ones8 = jnp.ones((K, DW), jnp.float32)
    zeros_deg = jnp.zeros((NP, DW), jnp.float32)
    zeros_acc = jnp.zeros((NP, DH), jnp.float32)

    deg_p = _sc_degree(dst, ones8, zeros_deg)
    xw1 = _tc_mm(x, W1)

    y1, dinv = _tc1(deg_p, deg_p, xw1)

    acc1 = _sc_aggregate(y1[:, :DH], y1[:, DH:], src, dst, zeros_acc)
    y2, xw2 = _tc_mid(acc1, acc1, xw1, dinv, b1.reshape(1, D), W2)

    acc2 = _sc_aggregate(y2[:, :DH], y2[:, DH:], src, dst, zeros_acc)
    return _tc_out(acc2, acc2, xw2, dinv,
                   b2.reshape(1, D), Wfc, bfc.reshape(1, _OUT))


# speedup vs baseline: 1.2072x; 1.0874x over previous
import functools

import jax
import jax.numpy as jnp
from jax import lax
from jax.experimental import pallas as pl
from jax.experimental.pallas import tpu as pltpu
from jax.experimental.pallas import tpu_sc as plsc

N = 10000
D = 128
E = 320000
NP = 10240
NC, NS = 2, 16
NW = NC * NS
EPT = E // NW
K = 80
CH = EPT // K
NB = 5
RPT = NP // NS
DW = 8

_MESH = plsc.VectorSubcoreMesh(
    core_axis_name="c", subcore_axis_name="s", num_cores=NC, num_subcores=NS)


@functools.partial(
    pl.kernel,
    out_type=jax.ShapeDtypeStruct((NC, NP, DW), jnp.float32),
    mesh=_MESH,
    compiler_params=pltpu.CompilerParams(use_tc_tiling_on_sc=False),
    scratch_types=[
        pltpu.VMEM((CH, K), jnp.int32),
        pltpu.VMEM((K, DW), jnp.float32),
        pltpu.VMEM_SHARED((NP, DW), jnp.float32),
    ],
)
def _sc_degree(dst_hbm, ones_hbm, zeros_hbm, out_hbm, idx_v, ones_v, deg_sh):
    c = lax.axis_index("c")
    s = lax.axis_index("s")
    wid = c * NS + s
    pltpu.sync_copy(zeros_hbm.at[pl.ds(s * RPT, RPT)],
                    deg_sh.at[pl.ds(s * RPT, RPT)])
    pltpu.sync_copy(dst_hbm.at[wid], idx_v)
    pltpu.sync_copy(ones_hbm, ones_v)
    plsc.subcore_barrier()

    @pl.loop(0, CH)
    def _chunk(j):
        pltpu.sync_copy(ones_v, deg_sh.at[idx_v.at[j]], add=True)

    plsc.subcore_barrier()
    pltpu.sync_copy(deg_sh.at[pl.ds(s * RPT, RPT)],
                    out_hbm.at[c, pl.ds(s * RPT, RPT)])


DH = D // 2


@functools.partial(
    pl.kernel,
    out_type=jax.ShapeDtypeStruct((NC, NP, D), jnp.float32),
    mesh=_MESH,
    compiler_params=pltpu.CompilerParams(use_tc_tiling_on_sc=False),
    scratch_types=(
        [
            pltpu.VMEM((CH, K), jnp.int32),
            pltpu.VMEM((CH, K), jnp.int32),
            pltpu.VMEM_SHARED((NP, DH), jnp.float32),
        ]
        + [pltpu.VMEM((K, DH), jnp.float32) for _ in range(NB)]
        + [pltpu.SemaphoreType.DMA for _ in range(2 * NB)]
    ),
)
def _sc_aggregate(y_hbm, src2_hbm, dst_hbm, zeros_hbm, out_hbm,
                  src_v, dst_v, acc_sh, *bufs_and_sems):
    bufs = bufs_and_sems[:NB]
    gsem = bufs_and_sems[NB:2 * NB]
    ssem = bufs_and_sems[2 * NB:]
    c = lax.axis_index("c")
    s = lax.axis_index("s")
    wid = c * NS + s
    pltpu.sync_copy(src2_hbm.at[wid], src_v)
    pltpu.sync_copy(dst_hbm.at[wid], dst_v)

    for half in (0, 1):
        if half == 1:
            @pl.loop(0, CH)
            def _bump(r):
                for q in range(K // 16):
                    src_v[r, pl.ds(q * 16, 16)] = (
                        src_v[r, pl.ds(q * 16, 16)] + 1)

        pltpu.sync_copy(zeros_hbm.at[pl.ds(s * RPT, RPT)],
                        acc_sh.at[pl.ds(s * RPT, RPT)])
        plsc.subcore_barrier()

        for b in range(NB):
            pltpu.async_copy(y_hbm.at[src_v.at[b]], bufs[b], gsem[b])

        @pl.loop(0, CH, step=NB)
        def _round(j):
            for b in range(NB):
                pltpu.make_async_copy(y_hbm.at[src_v.at[j + b]], bufs[b],
                                      gsem[b]).wait()
                pltpu.async_copy(bufs[b], acc_sh.at[dst_v.at[j + b]],
                                 ssem[b], add=True)
            for b in range(NB):
                @pl.when(j + b + NB < CH)
                def _refill():
                    pltpu.make_async_copy(bufs[b],
                                          acc_sh.at[dst_v.at[j + b]],
                                          ssem[b]).wait()
                    pltpu.async_copy(y_hbm.at[src_v.at[j + b + NB]],
                                     bufs[b], gsem[b])

        for b in range(NB):
            pltpu.make_async_copy(bufs[b], acc_sh.at[dst_v.at[CH - NB + b]],
                                  ssem[b]).wait()

        plsc.subcore_barrier()
        pltpu.sync_copy(acc_sh.at[pl.ds(s * RPT, RPT)],
                        out_hbm.at[c, pl.ds(s * RPT, RPT),
                                   pl.ds(half * DH, DH)])


_BLK = 2000


def _dot(a, b):
    return jnp.dot(a, b, preferred_element_type=jnp.float32,
                   precision=lax.Precision.HIGHEST)


def _tc_mm_body(x_ref, w1_ref, xw1_ref):
    xw1_ref[...] = _dot(x_ref[...], w1_ref[...])


def _tc1_body(degp_ref0, degp_ref1, xw_ref, y_ref, dinv_ref):
    deg = degp_ref0[0, :, 0:1] + degp_ref1[0, :, 0:1] + 1.0
    dinv = lax.rsqrt(deg)
    y_ref[...] = xw_ref[...] * dinv
    dinv_ref[...] = dinv


def _agg_h(a0_ref, a1_ref, xw_ref, dinv, b_ref):
    h = dinv * (a0_ref[0] + a1_ref[0]) + (dinv * dinv) * xw_ref[...]
    return jnp.maximum(h + b_ref[...], 0.0)


def _tc_mid_body(a0_ref, a1_ref, xw_ref, dinv_ref, b_ref, w_ref,
                 y_ref, xwn_ref):
    dinv = dinv_ref[...]
    h = _agg_h(a0_ref, a1_ref, xw_ref, dinv, b_ref)
    xwn = _dot(h, w_ref[...])
    xwn_ref[...] = xwn
    y_ref[...] = xwn * dinv


def _tc_out_body(a0_ref, a1_ref, xw_ref, dinv_ref, b_ref, wfc_ref, bfc_ref,
                 out_ref):
    dinv = dinv_ref[...]
    h = _agg_h(a0_ref, a1_ref, xw_ref, dinv, b_ref)
    out_ref[...] = _dot(h, wfc_ref[...]) + bfc_ref[...]


def _rows(i):
    return (i, 0)


def _fixed(i):
    return (0, 0)


def _degp_spec(core):
    return pl.BlockSpec((1, _BLK, DW), lambda i, c=core: (c, i, 0))


def _acc_spec(core):
    return pl.BlockSpec((1, _BLK, D), lambda i, c=core: (c, i, 0))


_tc_mm = pl.pallas_call(
    _tc_mm_body,
    grid=(N // _BLK,),
    in_specs=[
        pl.BlockSpec((_BLK, D), _rows),
        pl.BlockSpec((D, D), _fixed),
    ],
    out_specs=pl.BlockSpec((_BLK, D), _rows),
    out_shape=jax.ShapeDtypeStruct((N, D), jnp.float32),
)

_tc1 = pl.pallas_call(
    _tc1_body,
    grid=(N // _BLK,),
    in_specs=[
        _degp_spec(0),
        _degp_spec(1),
        pl.BlockSpec((_BLK, D), _rows),
    ],
    out_specs=[
        pl.BlockSpec((_BLK, D), _rows),
        pl.BlockSpec((_BLK, 1), _rows),
    ],
    out_shape=[
        jax.ShapeDtypeStruct((N, D), jnp.float32),
        jax.ShapeDtypeStruct((N, 1), jnp.float32),
    ],
)

_tc_mid = pl.pallas_call(
    _tc_mid_body,
    grid=(N // _BLK,),
    in_specs=[
        _acc_spec(0),
        _acc_spec(1),
        pl.BlockSpec((_BLK, D), _rows),
        pl.BlockSpec((_BLK, 1), _rows),
        pl.BlockSpec((1, D), _fixed),
        pl.BlockSpec((D, D), _fixed),
    ],
    out_specs=[
        pl.BlockSpec((_BLK, D), _rows),
        pl.BlockSpec((_BLK, D), _rows),
    ],
    out_shape=[
        jax.ShapeDtypeStruct((N, D), jnp.float32),
        jax.ShapeDtypeStruct((N, D), jnp.float32),
    ],
)

_OUT = 40

_tc_out = pl.pallas_call(
    _tc_out_body,
    grid=(N // _BLK,),
    in_specs=[
        _acc_spec(0),
        _acc_spec(1),
        pl.BlockSpec((_BLK, D), _rows),
        pl.BlockSpec((_BLK, 1), _rows),
        pl.BlockSpec((1, D), _fixed),
        pl.BlockSpec((D, _OUT), _fixed),
        pl.BlockSpec((1, _OUT), _fixed),
    ],
    out_specs=pl.BlockSpec((_BLK, _OUT), _rows),
    out_shape=jax.ShapeDtypeStruct((N, _OUT), jnp.float32),
)


def kernel(x, edge_index, W1, b1, W2, b2, Wfc, bfc):
    src2 = (edge_index[0].astype(jnp.int32) * 2).reshape(NW, CH, K)
    dst = edge_index[1].astype(jnp.int32).reshape(NW, CH, K)
    ones8 = jnp.ones((K, DW), jnp.float32)
    zeros_deg = jnp.zeros((NP, DW), jnp.float32)
    zeros_acc = jnp.zeros((NP, DH), jnp.float32)

    deg_p = _sc_degree(dst, ones8, zeros_deg)
    xw1 = _tc_mm(x, W1)

    y1, dinv = _tc1(deg_p, deg_p, xw1)

    acc1 = _sc_aggregate(y1.reshape(2 * N, DH), src2, dst, zeros_acc)
    y2, xw2 = _tc_mid(acc1, acc1, xw1, dinv, b1.reshape(1, D), W2)

    acc2 = _sc_aggregate(y2.reshape(2 * N, DH), src2, dst, zeros_acc)
    return _tc_out(acc2, acc2, xw2, dinv,
                   b2.reshape(1, D), Wfc, bfc.reshape(1, _OUT))


# speedup vs baseline: 1.2381x; 1.0256x over previous
import functools

import jax
import jax.numpy as jnp
from jax import lax
from jax.experimental import pallas as pl
from jax.experimental.pallas import tpu as pltpu
from jax.experimental.pallas import tpu_sc as plsc

N = 10000
D = 128
E = 320000
NP = 10240
NC, NS = 2, 16
NW = NC * NS
EPT = E // NW
K = 80
CH = EPT // K
NB = 5
RPT = NP // NS
DW = 8

_MESH = plsc.VectorSubcoreMesh(
    core_axis_name="c", subcore_axis_name="s", num_cores=NC, num_subcores=NS)


@functools.partial(
    pl.kernel,
    out_type=jax.ShapeDtypeStruct((NC, NP, DW), jnp.float32),
    mesh=_MESH,
    compiler_params=pltpu.CompilerParams(use_tc_tiling_on_sc=False),
    scratch_types=[
        pltpu.VMEM((CH, K), jnp.int32),
        pltpu.VMEM((K, DW), jnp.float32),
        pltpu.VMEM_SHARED((NP, DW), jnp.float32),
    ],
)
def _sc_degree(ei_hbm, ones_hbm, zeros_hbm, out_hbm, idx_v, ones_v, deg_sh):
    c = lax.axis_index("c")
    s = lax.axis_index("s")
    wid = c * NS + s
    pltpu.sync_copy(zeros_hbm.at[pl.ds(s * RPT, RPT)],
                    deg_sh.at[pl.ds(s * RPT, RPT)])
    pltpu.sync_copy(ei_hbm.at[1, wid], idx_v)
    pltpu.sync_copy(ones_hbm, ones_v)
    plsc.subcore_barrier()

    @pl.loop(0, CH)
    def _chunk(j):
        pltpu.sync_copy(ones_v, deg_sh.at[idx_v.at[j]], add=True)

    plsc.subcore_barrier()
    pltpu.sync_copy(deg_sh.at[pl.ds(s * RPT, RPT)],
                    out_hbm.at[c, pl.ds(s * RPT, RPT)])


DH = D // 2


@functools.partial(
    pl.kernel,
    out_type=jax.ShapeDtypeStruct((NC, NP, D), jnp.float32),
    mesh=_MESH,
    compiler_params=pltpu.CompilerParams(use_tc_tiling_on_sc=False),
    scratch_types=(
        [
            pltpu.VMEM((CH, K), jnp.int32),
            pltpu.VMEM((CH, K), jnp.int32),
            pltpu.VMEM_SHARED((NP, DH), jnp.float32),
        ]
        + [pltpu.VMEM((K, DH), jnp.float32) for _ in range(NB)]
        + [pltpu.SemaphoreType.DMA for _ in range(2 * NB)]
    ),
)
def _sc_aggregate(y_hbm, ei_hbm, zeros_hbm, out_hbm,
                  src_v, dst_v, acc_sh, *bufs_and_sems):
    bufs = bufs_and_sems[:NB]
    gsem = bufs_and_sems[NB:2 * NB]
    ssem = bufs_and_sems[2 * NB:]
    c = lax.axis_index("c")
    s = lax.axis_index("s")
    wid = c * NS + s
    pltpu.sync_copy(ei_hbm.at[0, wid], src_v)
    pltpu.sync_copy(ei_hbm.at[1, wid], dst_v)

    @pl.loop(0, CH)
    def _dbl(r):
        for q in range(K // 16):
            src_v[r, pl.ds(q * 16, 16)] = src_v[r, pl.ds(q * 16, 16)] * 2

    for half in (0, 1):
        if half == 1:
            @pl.loop(0, CH)
            def _bump(r):
                for q in range(K // 16):
                    src_v[r, pl.ds(q * 16, 16)] = (
                        src_v[r, pl.ds(q * 16, 16)] + 1)

        pltpu.sync_copy(zeros_hbm.at[pl.ds(s * RPT, RPT)],
                        acc_sh.at[pl.ds(s * RPT, RPT)])
        plsc.subcore_barrier()

        for b in range(NB):
            pltpu.async_copy(y_hbm.at[src_v.at[b]], bufs[b], gsem[b])

        @pl.loop(0, CH, step=NB)
        def _round(j):
            for b in range(NB):
                pltpu.make_async_copy(y_hbm.at[src_v.at[j + b]], bufs[b],
                                      gsem[b]).wait()
                pltpu.async_copy(bufs[b], acc_sh.at[dst_v.at[j + b]],
                                 ssem[b], add=True)
            for b in range(NB):
                @pl.when(j + b + NB < CH)
                def _refill():
                    pltpu.make_async_copy(bufs[b],
                                          acc_sh.at[dst_v.at[j + b]],
                                          ssem[b]).wait()
                    pltpu.async_copy(y_hbm.at[src_v.at[j + b + NB]],
                                     bufs[b], gsem[b])

        for b in range(NB):
            pltpu.make_async_copy(bufs[b], acc_sh.at[dst_v.at[CH - NB + b]],
                                  ssem[b]).wait()

        plsc.subcore_barrier()
        pltpu.sync_copy(acc_sh.at[pl.ds(s * RPT, RPT)],
                        out_hbm.at[c, pl.ds(s * RPT, RPT),
                                   pl.ds(half * DH, DH)])


_BLK = 2000


def _dot(a, b):
    return jnp.dot(a, b, preferred_element_type=jnp.float32,
                   precision=lax.Precision.HIGHEST)


def _tc_mm_body(x_ref, w1_ref, xw1_ref):
    xw1_ref[...] = _dot(x_ref[...], w1_ref[...])


def _tc1_body(degp_ref0, degp_ref1, xw_ref, y_ref, dinv_ref):
    deg = degp_ref0[0, :, 0:1] + degp_ref1[0, :, 0:1] + 1.0
    dinv = lax.rsqrt(deg)
    y_ref[...] = xw_ref[...] * dinv
    dinv_ref[...] = dinv


def _agg_h(a0_ref, a1_ref, xw_ref, dinv, b_ref):
    h = dinv * (a0_ref[0] + a1_ref[0]) + (dinv * dinv) * xw_ref[...]
    return jnp.maximum(h + b_ref[...], 0.0)


def _tc_mid_body(a0_ref, a1_ref, xw_ref, dinv_ref, b_ref, w_ref,
                 y_ref, xwn_ref):
    dinv = dinv_ref[...]
    h = _agg_h(a0_ref, a1_ref, xw_ref, dinv, b_ref)
    xwn = _dot(h, w_ref[...])
    xwn_ref[...] = xwn
    y_ref[...] = xwn * dinv


def _tc_out_body(a0_ref, a1_ref, xw_ref, dinv_ref, b_ref, wfc_ref, bfc_ref,
                 out_ref):
    dinv = dinv_ref[...]
    h = _agg_h(a0_ref, a1_ref, xw_ref, dinv, b_ref)
    out_ref[...] = _dot(h, wfc_ref[...]) + bfc_ref[...]


def _rows(i):
    return (i, 0)


def _fixed(i):
    return (0, 0)


def _degp_spec(core):
    return pl.BlockSpec((1, _BLK, DW), lambda i, c=core: (c, i, 0))


def _acc_spec(core):
    return pl.BlockSpec((1, _BLK, D), lambda i, c=core: (c, i, 0))


_tc_mm = pl.pallas_call(
    _tc_mm_body,
    grid=(N // _BLK,),
    in_specs=[
        pl.BlockSpec((_BLK, D), _rows),
        pl.BlockSpec((D, D), _fixed),
    ],
    out_specs=pl.BlockSpec((_BLK, D), _rows),
    out_shape=jax.ShapeDtypeStruct((N, D), jnp.float32),
)

_tc1 = pl.pallas_call(
    _tc1_body,
    grid=(N // _BLK,),
    in_specs=[
        _degp_spec(0),
        _degp_spec(1),
        pl.BlockSpec((_BLK, D), _rows),
    ],
    out_specs=[
        pl.BlockSpec((_BLK, D), _rows),
        pl.BlockSpec((_BLK, 1), _rows),
    ],
    out_shape=[
        jax.ShapeDtypeStruct((N, D), jnp.float32),
        jax.ShapeDtypeStruct((N, 1), jnp.float32),
    ],
)

_tc_mid = pl.pallas_call(
    _tc_mid_body,
    grid=(N // _BLK,),
    in_specs=[
        _acc_spec(0),
        _acc_spec(1),
        pl.BlockSpec((_BLK, D), _rows),
        pl.BlockSpec((_BLK, 1), _rows),
        pl.BlockSpec((1, D), _fixed),
        pl.BlockSpec((D, D), _fixed),
    ],
    out_specs=[
        pl.BlockSpec((_BLK, D), _rows),
        pl.BlockSpec((_BLK, D), _rows),
    ],
    out_shape=[
        jax.ShapeDtypeStruct((N, D), jnp.float32),
        jax.ShapeDtypeStruct((N, D), jnp.float32),
    ],
)

_OUT = 40

_tc_out = pl.pallas_call(
    _tc_out_body,
    grid=(N // _BLK,),
    in_specs=[
        _acc_spec(0),
        _acc_spec(1),
        pl.BlockSpec((_BLK, D), _rows),
        pl.BlockSpec((_BLK, 1), _rows),
        pl.BlockSpec((1, D), _fixed),
        pl.BlockSpec((D, _OUT), _fixed),
        pl.BlockSpec((1, _OUT), _fixed),
    ],
    out_specs=pl.BlockSpec((_BLK, _OUT), _rows),
    out_shape=jax.ShapeDtypeStruct((N, _OUT), jnp.float32),
)


def kernel(x, edge_index, W1, b1, W2, b2, Wfc, bfc):
    ei = edge_index.astype(jnp.int32).reshape(2, NW, CH, K)
    ones8 = jnp.ones((K, DW), jnp.float32)
    zeros_deg = jnp.zeros((NP, DW), jnp.float32)
    zeros_acc = jnp.zeros((NP, DH), jnp.float32)

    deg_p = _sc_degree(ei, ones8, zeros_deg)
    xw1 = _tc_mm(x, W1)

    y1, dinv = _tc1(deg_p, deg_p, xw1)

    acc1 = _sc_aggregate(y1.reshape(2 * N, DH), ei, zeros_acc)
    y2, xw2 = _tc_mid(acc1, acc1, xw1, dinv, b1.reshape(1, D), W2)

    acc2 = _sc_aggregate(y2.reshape(2 * N, DH), ei, zeros_acc)
    return _tc_out(acc2, acc2, xw2, dinv,
                   b2.reshape(1, D), Wfc, bfc.reshape(1, _OUT))
